# baseline (speedup 1.0000x reference)
import functools

import jax
import jax.numpy as jnp
from jax import lax
from jax.experimental import pallas as pl
from jax.experimental.pallas import tpu as pltpu
from jax.experimental.pallas import tpu_sc as plsc

N = 320000
D = 128
S = 512
NW = 32
NS = 16
SPW = S // NW
C = 200
NBUF = 4
L = 16
RPS = N // NS
OFFL = 528
SENT = N + 1


def _body(x_hbm, ids_hbm, out_hbm, ids_v, offp, offs_v, buf0, buf1, buf2,
          buf3, out_v, shared, shared_offs, sem0, sem1, sem2, sem3):
    cid = lax.axis_index("c")
    sid = lax.axis_index("s")
    wid = sid * 2 + cid
    seg_base = wid * SPW

    r0 = sid * RPS

    @pl.when(sid == 0)
    def _():
        ids_v[pl.ds(0, L)] = jnp.full((L,), -1, jnp.int32)
        pltpu.sync_copy(ids_hbm.at[pl.ds(0, RPS)], ids_v.at[pl.ds(16, RPS)])

    @pl.when(sid != 0)
    def _():
        src = pl.multiple_of(r0 - 16, 8)
        pltpu.sync_copy(ids_hbm.at[pl.ds(src, RPS + 16)], ids_v)

    full_sent = jnp.full((L,), SENT, jnp.int32)
    for j in range(OFFL // L):
        offp[pl.ds(j * L, L)] = full_sent

    iota = lax.iota(jnp.int32, L)

    def scan_body(i, carry):
        base = 16 + i * L
        v = ids_v[pl.ds(base, L)]
        pv = ids_v[pl.ds(base - 1, L)]
        val = jnp.full((L,), r0 + i * L + 1, jnp.int32) + iota
        plsc.store_scatter(offp, [v], val, mask=pv != v)
        return carry

    lax.fori_loop(0, RPS // L, scan_body, 0, unroll=4)

    pltpu.sync_copy(offp, shared.at[pl.ds(sid * OFFL, OFFL)])

    zero = jnp.zeros((L,), jnp.float32)
    for s in range(SPW):
        for j in range(D // L):
            out_v[s, pl.ds(j * L, L)] = zero

    plsc.subcore_barrier()

    @pl.when(sid == 0)
    def _():
        pltpu.sync_copy(shared, ids_v.at[pl.ds(0, NS * OFFL)])
        for j in range(OFFL // L):
            vals = [ids_v[pl.ds(t * OFFL + j * L, L)] for t in range(NS)]
            while len(vals) > 1:
                vals = [jnp.minimum(vals[i], vals[i + 1])
                        for i in range(0, len(vals), 2)]
            offp[pl.ds(j * L, L)] = vals[0]

        carry = SENT
        for j in range(OFFL // L - 1, -1, -1):
            v = offp[pl.ds(j * L, L)]
            sm = -lax.rev(plsc.cummax(lax.rev(-v, (0,))), (0,))
            vf = jnp.minimum(sm, jnp.full((L,), carry, jnp.int32))
            carry = vf[0]
            offp[pl.ds(j * L, L)] = vf - 1

        pltpu.sync_copy(offp, shared_offs)

    plsc.subcore_barrier()
    pltpu.sync_copy(shared_offs.at[pl.ds(seg_base, 24)], offs_v)

    ov0 = offs_v[pl.ds(0, L)]
    ov1 = offs_v[pl.ds(8, L)]
    o = [ov0[s] for s in range(L)] + [ov1[8]]
    lo_w = o[0]
    hi_w = o[SPW]

    CS = C - 8
    nchunks = lax.div(hi_w - lo_w + (CS - 1), CS)

    def chunk_start(k):
        aligned = jnp.bitwise_and(lo_w + k * CS, -8)
        return pl.multiple_of(jnp.minimum(aligned, N - C), 8)

    def issue(k, buf, sem):
        pltpu.make_async_copy(
            x_hbm.at[pl.ds(chunk_start(k), C)], buf, sem).start()

    def wait(k, buf, sem):
        pltpu.make_async_copy(
            x_hbm.at[pl.ds(chunk_start(k), C)], buf, sem).wait()

    def accumulate(k, buf):
        start = chunk_start(k)
        c_lo = lo_w + k * CS
        c_hi = c_lo + CS
        for s in range(SPW):
            s_lo = jnp.maximum(o[s], c_lo)
            s_hi = jnp.minimum(o[s + 1], c_hi)

            @pl.when(s_lo < s_hi)
            def _():
                accs = [out_v[s, pl.ds(j * L, L)] for j in range(D // L)]

                def row_body(r, accs):
                    return [a + buf[r, pl.ds(j * L, L)]
                            for j, a in enumerate(accs)]

                accs = lax.fori_loop(s_lo - start, s_hi - start, row_body,
                                     accs)
                for j in range(D // L):
                    out_v[s, pl.ds(j * L, L)] = accs[j]

    bufs = [buf0, buf1, buf2, buf3]
    sems = [sem0, sem1, sem2, sem3]

    for u in range(NBUF):
        @pl.when(u < nchunks)
        def _(u=u):
            issue(u, bufs[u], sems[u])

    def quad_body(q, carry):
        for u in range(NBUF):
            k = q * NBUF + u

            @pl.when(k < nchunks)
            def _(k=k, u=u):
                wait(k, bufs[u], sems[u])
                accumulate(k, bufs[u])

                @pl.when(k + NBUF < nchunks)
                def _(k=k, u=u):
                    issue(k + NBUF, bufs[u], sems[u])

        return carry

    nquads = lax.div(nchunks + (NBUF - 1), NBUF)
    lax.fori_loop(0, nquads, quad_body, 0)

    pltpu.sync_copy(out_v, out_hbm.at[pl.ds(seg_base, SPW)])


_fused = functools.partial(
    pl.kernel,
    out_type=jax.ShapeDtypeStruct((S, D), jnp.float32),
    mesh=plsc.VectorSubcoreMesh(core_axis_name="c", subcore_axis_name="s"),
    compiler_params=pltpu.CompilerParams(needs_layout_passes=False),
    scratch_types=[
        pltpu.VMEM((RPS + 16,), jnp.int32),
        pltpu.VMEM((OFFL,), jnp.int32),
        pltpu.VMEM((24,), jnp.int32),
        pltpu.VMEM((C, D), jnp.float32),
        pltpu.VMEM((C, D), jnp.float32),
        pltpu.VMEM((C, D), jnp.float32),
        pltpu.VMEM((C, D), jnp.float32),
        pltpu.VMEM((SPW, D), jnp.float32),
        pltpu.VMEM_SHARED((NS * OFFL,), jnp.int32),
        pltpu.VMEM_SHARED((OFFL,), jnp.int32),
        pltpu.SemaphoreType.DMA,
        pltpu.SemaphoreType.DMA,
        pltpu.SemaphoreType.DMA,
        pltpu.SemaphoreType.DMA,
    ],
)(_body)


def _mlp_body(p_ref, w1_ref, b1_ref, w2_ref, b2_ref, o_ref):
    h = jnp.dot(p_ref[...], w1_ref[...], preferred_element_type=jnp.float32)
    h = jnp.maximum(h + b1_ref[...], 0.0)
    o_ref[...] = (
        jnp.dot(h, w2_ref[...], preferred_element_type=jnp.float32)
        + b2_ref[...])


def _mlp(pooled, W1, b1, W2, b2):
    return pl.pallas_call(
        _mlp_body,
        out_shape=jax.ShapeDtypeStruct((S, 10), jnp.float32),
    )(pooled, W1, b1.reshape(1, -1), W2, b2.reshape(1, -1))


def kernel(x, batch, W1, b1, W2, b2):
    pooled = _fused(x, batch.astype(jnp.int32))
    return _mlp(pooled, W1, b1, W2, b2)


# speedup vs baseline: 1.2156x; 1.2156x over previous
import functools

import jax
import jax.numpy as jnp
from jax import lax
from jax.experimental import pallas as pl
from jax.experimental.pallas import tpu as pltpu
from jax.experimental.pallas import tpu_sc as plsc

N = 320000
D = 128
S = 512
NW = 32
NS = 16
SPW = S // NW
C = 384
L = 16
RPS = N // NS
OFFL = 528
SENT = N + 1


def _body(x_hbm, ids_hbm, out_hbm, ids_v, offp, offs_v, buf0, buf1,
          out_v, shared, shared_offs, sem0, sem1):
    cid = lax.axis_index("c")
    sid = lax.axis_index("s")
    wid = sid * 2 + cid
    seg_base = wid * SPW

    r0 = sid * RPS

    @pl.when(sid == 0)
    def _():
        ids_v[pl.ds(0, L)] = jnp.full((L,), -1, jnp.int32)
        pltpu.sync_copy(ids_hbm.at[pl.ds(0, RPS)], ids_v.at[pl.ds(16, RPS)])

    @pl.when(sid != 0)
    def _():
        src = pl.multiple_of(r0 - 16, 8)
        pltpu.sync_copy(ids_hbm.at[pl.ds(src, RPS + 16)], ids_v)

    full_sent = jnp.full((L,), SENT, jnp.int32)
    for j in range(OFFL // L):
        offp[pl.ds(j * L, L)] = full_sent

    iota = lax.iota(jnp.int32, L)

    def scan_body(i, carry):
        base = 16 + i * L
        v = ids_v[pl.ds(base, L)]
        pv = ids_v[pl.ds(base - 1, L)]
        val = jnp.full((L,), r0 + i * L + 1, jnp.int32) + iota
        plsc.store_scatter(offp, [v], val, mask=pv != v)
        return carry

    lax.fori_loop(0, RPS // L, scan_body, 0, unroll=4)

    pltpu.sync_copy(offp, shared.at[pl.ds(sid * OFFL, OFFL)])

    zero = jnp.zeros((L,), jnp.float32)
    for s in range(SPW):
        for j in range(D // L):
            out_v[s, pl.ds(j * L, L)] = zero

    plsc.subcore_barrier()

    @pl.when(sid == 0)
    def _():
        pltpu.sync_copy(shared, ids_v.at[pl.ds(0, NS * OFFL)])
        for j in range(OFFL // L):
            vals = [ids_v[pl.ds(t * OFFL + j * L, L)] for t in range(NS)]
            while len(vals) > 1:
                vals = [jnp.minimum(vals[i], vals[i + 1])
                        for i in range(0, len(vals), 2)]
            offp[pl.ds(j * L, L)] = vals[0]

        carry = SENT
        for j in range(OFFL // L - 1, -1, -1):
            v = offp[pl.ds(j * L, L)]
            sm = -lax.rev(plsc.cummax(lax.rev(-v, (0,))), (0,))
            vf = jnp.minimum(sm, jnp.full((L,), carry, jnp.int32))
            carry = vf[0]
            offp[pl.ds(j * L, L)] = vf - 1

        pltpu.sync_copy(offp, shared_offs)

    plsc.subcore_barrier()
    pltpu.sync_copy(shared_offs.at[pl.ds(seg_base, 24)], offs_v)

    ov0 = offs_v[pl.ds(0, L)]
    ov1 = offs_v[pl.ds(8, L)]
    o = [ov0[s] for s in range(L)] + [ov1[8]]
    lo_w = o[0]
    hi_w = o[SPW]

    CS = C - 8
    nchunks = lax.div(hi_w - lo_w + (CS - 1), CS)

    def chunk_start(k):
        aligned = jnp.bitwise_and(lo_w + k * CS, -8)
        return pl.multiple_of(jnp.minimum(aligned, N - C), 8)

    H = C // 2

    def issue(k, buf, sem):
        s0 = chunk_start(k)
        s1 = pl.multiple_of(s0 + H, 8)
        pltpu.make_async_copy(
            x_hbm.at[pl.ds(s0, H)], buf.at[pl.ds(0, H)], sem).start()
        pltpu.make_async_copy(
            x_hbm.at[pl.ds(s1, H)], buf.at[pl.ds(H, H)], sem).start()

    def wait(k, buf, sem):
        s0 = chunk_start(k)
        s1 = pl.multiple_of(s0 + H, 8)
        pltpu.make_async_copy(
            x_hbm.at[pl.ds(s0, H)], buf.at[pl.ds(0, H)], sem).wait()
        pltpu.make_async_copy(
            x_hbm.at[pl.ds(s1, H)], buf.at[pl.ds(H, H)], sem).wait()

    def accumulate(k, buf):
        start = chunk_start(k)
        c_lo = lo_w + k * CS
        c_hi = c_lo + CS
        for s in range(SPW):
            s_lo = jnp.maximum(o[s], c_lo)
            s_hi = jnp.minimum(o[s + 1], c_hi)

            @pl.when(s_lo < s_hi)
            def _():
                accs = [out_v[s, pl.ds(j * L, L)] for j in range(D // L)]

                def row_body(r, accs):
                    return [a + buf[r, pl.ds(j * L, L)]
                            for j, a in enumerate(accs)]

                accs = lax.fori_loop(s_lo - start, s_hi - start, row_body,
                                     accs)
                for j in range(D // L):
                    out_v[s, pl.ds(j * L, L)] = accs[j]

    @pl.when(nchunks > 0)
    def _():
        issue(0, buf0, sem0)

    @pl.when(nchunks > 1)
    def _():
        issue(1, buf1, sem1)

    def pair_body(p, carry):
        k0 = 2 * p
        k1 = k0 + 1

        wait(k0, buf0, sem0)
        accumulate(k0, buf0)

        @pl.when(k0 + 2 < nchunks)
        def _():
            issue(k0 + 2, buf0, sem0)

        @pl.when(k1 < nchunks)
        def _():
            wait(k1, buf1, sem1)
            accumulate(k1, buf1)

            @pl.when(k1 + 2 < nchunks)
            def _():
                issue(k1 + 2, buf1, sem1)

        return carry

    npairs = lax.div(nchunks + 1, 2)
    lax.fori_loop(0, npairs, pair_body, 0)

    pltpu.sync_copy(out_v, out_hbm.at[pl.ds(seg_base, SPW)])


_fused = functools.partial(
    pl.kernel,
    out_type=jax.ShapeDtypeStruct((S, D), jnp.float32),
    mesh=plsc.VectorSubcoreMesh(core_axis_name="c", subcore_axis_name="s"),
    compiler_params=pltpu.CompilerParams(needs_layout_passes=False),
    scratch_types=[
        pltpu.VMEM((RPS + 16,), jnp.int32),
        pltpu.VMEM((OFFL,), jnp.int32),
        pltpu.VMEM((24,), jnp.int32),
        pltpu.VMEM((C, D), jnp.float32),
        pltpu.VMEM((C, D), jnp.float32),
        pltpu.VMEM((SPW, D), jnp.float32),
        pltpu.VMEM_SHARED((NS * OFFL,), jnp.int32),
        pltpu.VMEM_SHARED((OFFL,), jnp.int32),
        pltpu.SemaphoreType.DMA,
        pltpu.SemaphoreType.DMA,
    ],
)(_body)


def _mlp_body(p_ref, w1_ref, b1_ref, w2_ref, b2_ref, o_ref):
    h = jnp.dot(p_ref[...], w1_ref[...], preferred_element_type=jnp.float32)
    h = jnp.maximum(h + b1_ref[...], 0.0)
    o_ref[...] = (
        jnp.dot(h, w2_ref[...], preferred_element_type=jnp.float32)
        + b2_ref[...])


def _mlp(pooled, W1, b1, W2, b2):
    return pl.pallas_call(
        _mlp_body,
        out_shape=jax.ShapeDtypeStruct((S, 10), jnp.float32),
    )(pooled, W1, b1.reshape(1, -1), W2, b2.reshape(1, -1))


def kernel(x, batch, W1, b1, W2, b2):
    pooled = _fused(x, batch.astype(jnp.int32))
    return _mlp(pooled, W1, b1, W2, b2)


# speedup vs baseline: 1.2219x; 1.0052x over previous
import functools

import jax
import jax.numpy as jnp
from jax import lax
from jax.experimental import pallas as pl
from jax.experimental.pallas import tpu as pltpu
from jax.experimental.pallas import tpu_sc as plsc

N = 320000
D = 128
S = 512
NW = 32
NS = 16
SPW = S // NW
C = 384
L = 16
RPS = N // NS
OFFL = 528
SENT = N + 1


def _body(x_hbm, ids_hbm, out_hbm, ids_v, offp, offs_v, buf0, buf1,
          out_v, shared, shared_offs, sem0, sem1):
    cid = lax.axis_index("c")
    sid = lax.axis_index("s")
    wid = sid * 2 + cid
    seg_base = wid * SPW

    r0 = sid * RPS

    @pl.when(sid == 0)
    def _():
        ids_v[pl.ds(0, L)] = jnp.full((L,), -1, jnp.int32)
        pltpu.sync_copy(ids_hbm.at[pl.ds(0, RPS)], ids_v.at[pl.ds(16, RPS)])

    @pl.when(sid != 0)
    def _():
        src = pl.multiple_of(r0 - 16, 8)
        pltpu.sync_copy(ids_hbm.at[pl.ds(src, RPS + 16)], ids_v)

    full_sent = jnp.full((L,), SENT, jnp.int32)
    for j in range(OFFL // L):
        offp[pl.ds(j * L, L)] = full_sent

    iota = lax.iota(jnp.int32, L)

    def scan_body(i, carry):
        base = 16 + i * L
        v = ids_v[pl.ds(base, L)]
        pv = ids_v[pl.ds(base - 1, L)]
        val = jnp.full((L,), r0 + i * L + 1, jnp.int32) + iota
        plsc.store_scatter(offp, [v], val, mask=pv != v)
        return carry

    lax.fori_loop(0, RPS // L, scan_body, 0, unroll=4)

    pltpu.sync_copy(offp, shared.at[pl.ds(sid * OFFL, OFFL)])

    zero = jnp.zeros((L,), jnp.float32)
    for s in range(SPW):
        for j in range(D // L):
            out_v[s, pl.ds(j * L, L)] = zero

    plsc.subcore_barrier()

    @pl.when(sid == 0)
    def _():
        pltpu.sync_copy(shared, ids_v.at[pl.ds(0, NS * OFFL)])
        for j in range(OFFL // L):
            vals = [ids_v[pl.ds(t * OFFL + j * L, L)] for t in range(NS)]
            while len(vals) > 1:
                vals = [jnp.minimum(vals[i], vals[i + 1])
                        for i in range(0, len(vals), 2)]
            offp[pl.ds(j * L, L)] = vals[0]

        carry = SENT
        for j in range(OFFL // L - 1, -1, -1):
            v = offp[pl.ds(j * L, L)]
            sm = -lax.rev(plsc.cummax(lax.rev(-v, (0,))), (0,))
            vf = jnp.minimum(sm, jnp.full((L,), carry, jnp.int32))
            carry = vf[0]
            offp[pl.ds(j * L, L)] = vf - 1

        pltpu.sync_copy(offp, shared_offs)

    plsc.subcore_barrier()
    pltpu.sync_copy(shared_offs.at[pl.ds(seg_base, 24)], offs_v)

    ov0 = offs_v[pl.ds(0, L)]
    ov1 = offs_v[pl.ds(8, L)]
    o = [ov0[s] for s in range(L)] + [ov1[8]]
    lo_w = o[0]
    hi_w = o[SPW]

    CS = C - 8
    nchunks = lax.div(hi_w - lo_w + (CS - 1), CS)

    def chunk_start(k):
        aligned = jnp.bitwise_and(lo_w + k * CS, -8)
        return pl.multiple_of(jnp.minimum(aligned, N - C), 8)

    def issue(k, buf, sem):
        pltpu.make_async_copy(
            x_hbm.at[pl.ds(chunk_start(k), C)], buf, sem).start()

    def wait(k, buf, sem):
        pltpu.make_async_copy(
            x_hbm.at[pl.ds(chunk_start(k), C)], buf, sem).wait()

    def accumulate(k, buf):
        start = chunk_start(k)
        c_lo = lo_w + k * CS
        c_hi = c_lo + CS
        for s in range(SPW):
            s_lo = jnp.maximum(o[s], c_lo)
            s_hi = jnp.minimum(o[s + 1], c_hi)

            @pl.when(s_lo < s_hi)
            def _():
                accs = [out_v[s, pl.ds(j * L, L)] for j in range(D // L)]

                def row_body(r, accs):
                    return [a + buf[r, pl.ds(j * L, L)]
                            for j, a in enumerate(accs)]

                accs = lax.fori_loop(s_lo - start, s_hi - start, row_body,
                                     accs)
                for j in range(D // L):
                    out_v[s, pl.ds(j * L, L)] = accs[j]

    @pl.when(nchunks > 0)
    def _():
        issue(0, buf0, sem0)

    @pl.when(nchunks > 1)
    def _():
        issue(1, buf1, sem1)

    def pair_body(p, carry):
        k0 = 2 * p
        k1 = k0 + 1

        wait(k0, buf0, sem0)
        accumulate(k0, buf0)

        @pl.when(k0 + 2 < nchunks)
        def _():
            issue(k0 + 2, buf0, sem0)

        @pl.when(k1 < nchunks)
        def _():
            wait(k1, buf1, sem1)
            accumulate(k1, buf1)

            @pl.when(k1 + 2 < nchunks)
            def _():
                issue(k1 + 2, buf1, sem1)

        return carry

    npairs = lax.div(nchunks + 1, 2)
    lax.fori_loop(0, npairs, pair_body, 0)

    pltpu.sync_copy(out_v, out_hbm.at[pl.ds(seg_base, SPW)])


_fused = functools.partial(
    pl.kernel,
    out_type=jax.ShapeDtypeStruct((S, D), jnp.float32),
    mesh=plsc.VectorSubcoreMesh(core_axis_name="c", subcore_axis_name="s"),
    compiler_params=pltpu.CompilerParams(needs_layout_passes=False),
    scratch_types=[
        pltpu.VMEM((RPS + 16,), jnp.int32),
        pltpu.VMEM((OFFL,), jnp.int32),
        pltpu.VMEM((24,), jnp.int32),
        pltpu.VMEM((C, D), jnp.float32),
        pltpu.VMEM((C, D), jnp.float32),
        pltpu.VMEM((SPW, D), jnp.float32),
        pltpu.VMEM_SHARED((NS * OFFL,), jnp.int32),
        pltpu.VMEM_SHARED((OFFL,), jnp.int32),
        pltpu.SemaphoreType.DMA,
        pltpu.SemaphoreType.DMA,
    ],
)(_body)


def _mlp_body(p_ref, w1_ref, b1_ref, w2_ref, b2_ref, o_ref):
    h = jnp.dot(p_ref[...], w1_ref[...], preferred_element_type=jnp.float32)
    h = jnp.maximum(h + b1_ref[...], 0.0)
    o_ref[...] = (
        jnp.dot(h, w2_ref[...], preferred_element_type=jnp.float32)
        + b2_ref[...])


def _mlp(pooled, W1, b1, W2, b2):
    return pl.pallas_call(
        _mlp_body,
        out_shape=jax.ShapeDtypeStruct((S, 10), jnp.float32),
    )(pooled, W1, b1.reshape(1, -1), W2, b2.reshape(1, -1))


def kernel(x, batch, W1, b1, W2, b2):
    pooled = _fused(x, batch.astype(jnp.int32))
    return _mlp(pooled, W1, b1, W2, b2)
